# Initial kernel scaffold; baseline (speedup 1.0000x reference)
import jax
import jax.numpy as jnp
from jax.experimental import pallas as pl


def kernel(x, edge_index, W, b):
    raise NotImplementedError("write your pallas kernel here")


# speedup vs baseline: 2.0205x; 2.0205x over previous
import functools

import jax
import jax.numpy as jnp
from jax import lax
from jax.experimental import pallas as pl
from jax.experimental.pallas import tpu as pltpu
from jax.experimental.pallas import tpu_sc as plsc

N = 10000
D = 128
NC = 2
NS = 16
NW = NC * NS
NPB = 320
NPAD = NW * NPB
TRASH = NPAD
NALL = NPAD + 256
ZPW = NALL // NS
HALF = NPAD // 2
HALL = HALF + 128
HZPW = HALL // NS

E = 320000
GW = 512
NGRP = 640
EPAD = NGRP * GW
GPS = NGRP // NS
NB = GPS // 2

_f32 = jnp.float32


@functools.cache
def _mesh():
    return plsc.VectorSubcoreMesh(
        core_axis_name="c", subcore_axis_name="s", num_cores=NC, num_subcores=NS)


def _fill(ref, n, value):
    v = jnp.full((16,), value, _f32)
    for q in range(n // 16):
        ref[pl.ds(q * 16, 16)] = v


def _deg_body(dst_hbm, deg_hbm, deg_sp, dstv, onesv, zbuf, degv):
    cid = lax.axis_index("c")
    sid = lax.axis_index("s")
    gw = sid * NC + cid

    _fill(onesv, GW, 1.0)
    _fill(zbuf, ZPW, 0.0)
    pltpu.sync_copy(zbuf, deg_sp.at[pl.ds(sid * ZPW, ZPW)])
    pltpu.sync_copy(dst_hbm.at[pl.ds(sid * GPS, GPS)], dstv)
    plsc.subcore_barrier()

    def deg_body(j, carry):
        pltpu.sync_copy(onesv, deg_sp.at[dstv.at[j, 0]], add=True)
        return carry

    lax.fori_loop(0, GPS, deg_body, 0)
    plsc.subcore_barrier()
    pltpu.sync_copy(deg_sp.at[pl.ds(gw * NPB, NPB)], degv)
    pltpu.sync_copy(degv, deg_hbm.at[pl.ds(gw * NPB, NPB)])


@functools.cache
def _deg():
    return pl.kernel(
        _deg_body,
        out_type=jax.ShapeDtypeStruct((NPAD,), _f32),
        mesh=_mesh(),
        scratch_types=[
            pltpu.VMEM_SHARED((NALL,), _f32),
            pltpu.VMEM((GPS, 1, GW), jnp.int32),
            pltpu.VMEM((GW,), _f32),
            pltpu.VMEM((ZPW,), _f32),
            pltpu.VMEM((NPB,), _f32),
        ],
    )


def _prop_body(src_hbm, dst_hbm, h_hbm, agg_hbm, agg_sp, srcv, dstv, rows,
               zrows):
    cid = lax.axis_index("c")
    sid = lax.axis_index("s")
    gw = sid * NC + cid

    for r in range(8):
        for q in range(D // 16):
            zrows[r, pl.ds(q * 16, 16)] = jnp.zeros((16,), _f32)
    for t in range(HZPW // 8):
        pltpu.sync_copy(zrows, agg_sp.at[pl.ds(sid * HZPW + t * 8, 8)])
    plsc.subcore_barrier()

    lo = cid * HALF
    trash = jnp.full((16,), HALF, jnp.int32) + sid.astype(jnp.int32)

    for p in range(2):
        boff = sid * GPS + p * NB
        pltpu.sync_copy(src_hbm.at[pl.ds(boff, NB)], srcv)
        pltpu.sync_copy(dst_hbm.at[pl.ds(boff, NB)], dstv)

        def remap_body(j, carry):
            for q in range(GW // 16):
                d = dstv[j, 0, pl.ds(q * 16, 16)] - lo
                keep = (d >= 0) & (d < HALF)
                dstv[j, 0, pl.ds(q * 16, 16)] = jnp.where(keep, d, trash)
            return carry

        lax.fori_loop(0, NB, remap_body, 0)

        def body(j, carry):
            pltpu.sync_copy(h_hbm.at[srcv.at[j, 0]], rows)
            pltpu.sync_copy(rows, agg_sp.at[dstv.at[j, 0]], add=True)
            return carry

        lax.fori_loop(0, NB, body, 0)
    plsc.subcore_barrier()
    base = sid * (HALF // NS)
    pltpu.sync_copy(agg_sp.at[pl.ds(base, HALF // NS)],
                    rows.at[pl.ds(0, HALF // NS)])
    pltpu.sync_copy(rows.at[pl.ds(0, HALF // NS)],
                    agg_hbm.at[pl.ds(cid * HALF + base, HALF // NS)])


@functools.cache
def _prop():
    return pl.kernel(
        _prop_body,
        out_type=jax.ShapeDtypeStruct((NPAD, D), _f32),
        mesh=_mesh(),
        scratch_types=[
            pltpu.VMEM_SHARED((HALL, D), _f32),
            pltpu.VMEM((NB, 1, GW), jnp.int32),
            pltpu.VMEM((NB, 1, GW), jnp.int32),
            pltpu.VMEM((GW, D), _f32),
            pltpu.VMEM((8, D), _f32),
        ],
    )


TBLK = 1024


def _t1_body(deg_ref, x_ref, dinv_ref, h1_ref):
    dinv = lax.rsqrt(jnp.maximum(deg_ref[...], 1.0))
    dinv_ref[...] = dinv
    h1_ref[...] = x_ref[...] * dinv


def _t1(deg2d, xp):
    return pl.pallas_call(
        _t1_body,
        grid=(NPAD // TBLK,),
        in_specs=[pl.BlockSpec((TBLK, 1), lambda i: (i, 0)),
                  pl.BlockSpec((TBLK, D), lambda i: (i, 0))],
        out_specs=[pl.BlockSpec((TBLK, 1), lambda i: (i, 0)),
                   pl.BlockSpec((TBLK, D), lambda i: (i, 0))],
        out_shape=(jax.ShapeDtypeStruct((NPAD, 1), _f32),
                   jax.ShapeDtypeStruct((NPAD, D), _f32)),
    )(deg2d, xp)


def _t2_body(p_ref, dinv_ref, x1_ref, h2_ref):
    dinv = dinv_ref[...]
    x1 = p_ref[...] * (-dinv)
    x1_ref[...] = x1
    h2_ref[...] = x1 * dinv


def _t2(p, dinv2d):
    return pl.pallas_call(
        _t2_body,
        grid=(NPAD // TBLK,),
        in_specs=[pl.BlockSpec((TBLK, D), lambda i: (i, 0)),
                  pl.BlockSpec((TBLK, 1), lambda i: (i, 0))],
        out_specs=[pl.BlockSpec((TBLK, D), lambda i: (i, 0)),
                   pl.BlockSpec((TBLK, D), lambda i: (i, 0))],
        out_shape=(jax.ShapeDtypeStruct((NPAD, D), _f32),
                   jax.ShapeDtypeStruct((NPAD, D), _f32)),
    )(p, dinv2d)


BLK = 512


def _mm_body(x_ref, x1_ref, p_ref, dinv_ref, w0_ref, w1_ref, w2_ref,
             b_ref, o_ref):
    x = x_ref[...]
    x2 = p_ref[...] * (-2.0 * dinv_ref[...]) - x
    acc = jnp.dot(x, w0_ref[...], preferred_element_type=_f32)
    acc += jnp.dot(x1_ref[...], w1_ref[...], preferred_element_type=_f32)
    acc += jnp.dot(x2, w2_ref[...], preferred_element_type=_f32)
    o_ref[...] = jnp.maximum(acc + b_ref[...], 0.0)


def _matmul(xp, x1, p, dinv2d, W, b):
    w0, w1, w2 = W[:D], W[D:2 * D], W[2 * D:]
    return pl.pallas_call(
        _mm_body,
        grid=(NPAD // BLK,),
        in_specs=[pl.BlockSpec((BLK, D), lambda i: (i, 0))] * 3
        + [pl.BlockSpec((BLK, 1), lambda i: (i, 0))]
        + [pl.BlockSpec((D, D), lambda i: (0, 0))] * 3
        + [pl.BlockSpec((1, D), lambda i: (0, 0))],
        out_specs=pl.BlockSpec((BLK, D), lambda i: (i, 0)),
        out_shape=jax.ShapeDtypeStruct((NPAD, D), _f32),
    )(xp, x1, p, dinv2d, w0, w1, w2, b.reshape(1, D))


def kernel(x, edge_index, W, b):
    src = edge_index[0].astype(jnp.int32)
    dst = edge_index[1].astype(jnp.int32)
    src_p = jnp.concatenate(
        [src, jnp.zeros((EPAD - E,), jnp.int32)]).reshape(NGRP, 1, GW)
    dst_p = jnp.concatenate(
        [dst, jnp.full((EPAD - E,), TRASH, jnp.int32)]).reshape(NGRP, 1, GW)
    xp = jnp.pad(x, ((0, NPAD - N), (0, 0)))

    deg = _deg()(dst_p)
    dinv2d, h1 = _t1(deg.reshape(NPAD, 1), xp)
    agg1 = _prop()(src_p, dst_p, h1)
    x1, h2 = _t2(agg1, dinv2d)
    agg2 = _prop()(src_p, dst_p, h2)
    out = _matmul(xp, x1, agg2, dinv2d, W, b)
    return out[:N]


# speedup vs baseline: 2.8080x; 1.3898x over previous
import functools

import jax
import jax.numpy as jnp
from jax import lax
from jax.experimental import pallas as pl
from jax.experimental.pallas import tpu as pltpu
from jax.experimental.pallas import tpu_sc as plsc

N = 10000
D = 128
NC = 2
NS = 16
NW = NC * NS
NPB = 320
NPAD = NW * NPB
TRASH = NPAD
NALL = NPAD + 256
ZPW = NALL // NS
HALF = NPAD // 2
HALL = HALF + 128
HZPW = HALL // NS

E = 320000
CHUNK = 128
CPW = 79
NCHUNKS = NW * CPW
EPAD = NCHUNKS * CHUNK
CPW_DEG = NCHUNKS // NS

_f32 = jnp.float32


@functools.cache
def _mesh():
    return plsc.VectorSubcoreMesh(
        core_axis_name="c", subcore_axis_name="s", num_cores=NC, num_subcores=NS)


def _fill(ref, n, value):
    v = jnp.full((16,), value, _f32)
    for q in range(n // 16):
        ref[pl.ds(q * 16, 16)] = v


def _deg_body(dst_hbm, deg_hbm, deg_sp, dstv, onesv, zbuf, degv):
    cid = lax.axis_index("c")
    sid = lax.axis_index("s")
    gw = sid * NC + cid

    _fill(onesv, CHUNK, 1.0)
    _fill(zbuf, ZPW, 0.0)
    pltpu.sync_copy(zbuf, deg_sp.at[pl.ds(sid * ZPW, ZPW)])
    pltpu.sync_copy(dst_hbm.at[pl.ds(sid * CPW_DEG, CPW_DEG)], dstv)
    plsc.subcore_barrier()

    def deg_body(j, carry):
        pltpu.sync_copy(onesv, deg_sp.at[dstv.at[j, 0]], add=True)
        return carry

    lax.fori_loop(0, CPW_DEG, deg_body, 0)
    plsc.subcore_barrier()
    pltpu.sync_copy(deg_sp.at[pl.ds(gw * NPB, NPB)], degv)
    pltpu.sync_copy(degv, deg_hbm.at[pl.ds(gw * NPB, NPB)])


@functools.cache
def _deg():
    return pl.kernel(
        _deg_body,
        out_type=jax.ShapeDtypeStruct((NPAD,), _f32),
        mesh=_mesh(),
        scratch_types=[
            pltpu.VMEM_SHARED((NALL,), _f32),
            pltpu.VMEM((CPW_DEG, 1, CHUNK), jnp.int32),
            pltpu.VMEM((CHUNK,), _f32),
            pltpu.VMEM((ZPW,), _f32),
            pltpu.VMEM((NPB,), _f32),
        ],
    )


def _prop_body(src_hbm, dst_hbm, h_hbm, agg_hbm, agg_sp, srcv, dstv, rows,
               zrows):
    cid = lax.axis_index("c")
    sid = lax.axis_index("s")
    gw = sid * NC + cid

    for r in range(HZPW):
        for q in range(D // 16):
            zrows[r, pl.ds(q * 16, 16)] = jnp.zeros((16,), _f32)
    pltpu.sync_copy(zrows, agg_sp.at[pl.ds(sid * HZPW, HZPW)])
    plsc.subcore_barrier()

    lo = cid * HALF
    trash = jnp.full((16,), HALF, jnp.int32) + sid.astype(jnp.int32)

    for p in range(2):
        boff = sid * CPW_DEG + p * CPW
        pltpu.sync_copy(src_hbm.at[pl.ds(boff, CPW)], srcv)
        pltpu.sync_copy(dst_hbm.at[pl.ds(boff, CPW)], dstv)

        def remap_body(j, carry):
            for q in range(CHUNK // 16):
                d = dstv[j, 0, pl.ds(q * 16, 16)] - lo
                keep = (d >= 0) & (d < HALF)
                dstv[j, 0, pl.ds(q * 16, 16)] = jnp.where(keep, d, trash)
            return carry

        lax.fori_loop(0, CPW, remap_body, 0)

        def body(j, carry):
            pltpu.sync_copy(h_hbm.at[srcv.at[j, 0]], rows)
            pltpu.sync_copy(rows, agg_sp.at[dstv.at[j, 0]], add=True)
            return carry

        lax.fori_loop(0, CPW, body, 0)
    plsc.subcore_barrier()
    pltpu.sync_copy(agg_sp.at[pl.ds(sid * (HALF // NS), HALF // NS)],
                    zrows.at[pl.ds(0, HALF // NS)])
    pltpu.sync_copy(zrows.at[pl.ds(0, HALF // NS)],
                    agg_hbm.at[pl.ds(cid * HALF + sid * (HALF // NS),
                                     HALF // NS)])


@functools.cache
def _prop():
    return pl.kernel(
        _prop_body,
        out_type=jax.ShapeDtypeStruct((NPAD, D), _f32),
        mesh=_mesh(),
        scratch_types=[
            pltpu.VMEM_SHARED((HALL, D), _f32),
            pltpu.VMEM((CPW, 1, CHUNK), jnp.int32),
            pltpu.VMEM((CPW, 1, CHUNK), jnp.int32),
            pltpu.VMEM((CHUNK, D), _f32),
            pltpu.VMEM((HZPW, D), _f32),
        ],
    )


TBLK = 1024


def _t1_body(deg_ref, x_ref, dinv_ref, h1_ref):
    dinv = lax.rsqrt(jnp.maximum(deg_ref[...], 1.0))
    dinv_ref[...] = dinv
    h1_ref[...] = x_ref[...] * dinv


def _t1(deg2d, xp):
    return pl.pallas_call(
        _t1_body,
        grid=(NPAD // TBLK,),
        in_specs=[pl.BlockSpec((TBLK, 1), lambda i: (i, 0)),
                  pl.BlockSpec((TBLK, D), lambda i: (i, 0))],
        out_specs=[pl.BlockSpec((TBLK, 1), lambda i: (i, 0)),
                   pl.BlockSpec((TBLK, D), lambda i: (i, 0))],
        out_shape=(jax.ShapeDtypeStruct((NPAD, 1), _f32),
                   jax.ShapeDtypeStruct((NPAD, D), _f32)),
    )(deg2d, xp)


def _t2_body(p_ref, dinv_ref, x1_ref, h2_ref):
    dinv = dinv_ref[...]
    x1 = p_ref[...] * (-dinv)
    x1_ref[...] = x1
    h2_ref[...] = x1 * dinv


def _t2(p, dinv2d):
    return pl.pallas_call(
        _t2_body,
        grid=(NPAD // TBLK,),
        in_specs=[pl.BlockSpec((TBLK, D), lambda i: (i, 0)),
                  pl.BlockSpec((TBLK, 1), lambda i: (i, 0))],
        out_specs=[pl.BlockSpec((TBLK, D), lambda i: (i, 0)),
                   pl.BlockSpec((TBLK, D), lambda i: (i, 0))],
        out_shape=(jax.ShapeDtypeStruct((NPAD, D), _f32),
                   jax.ShapeDtypeStruct((NPAD, D), _f32)),
    )(p, dinv2d)


BLK = 512


def _mm_body(x_ref, x1_ref, p_ref, dinv_ref, w0_ref, w1_ref, w2_ref,
             b_ref, o_ref):
    x = x_ref[...]
    x2 = p_ref[...] * (-2.0 * dinv_ref[...]) - x
    acc = jnp.dot(x, w0_ref[...], preferred_element_type=_f32)
    acc += jnp.dot(x1_ref[...], w1_ref[...], preferred_element_type=_f32)
    acc += jnp.dot(x2, w2_ref[...], preferred_element_type=_f32)
    o_ref[...] = jnp.maximum(acc + b_ref[...], 0.0)


def _matmul(xp, x1, p, dinv2d, W, b):
    w0, w1, w2 = W[:D], W[D:2 * D], W[2 * D:]
    return pl.pallas_call(
        _mm_body,
        grid=(NPAD // BLK,),
        in_specs=[pl.BlockSpec((BLK, D), lambda i: (i, 0))] * 3
        + [pl.BlockSpec((BLK, 1), lambda i: (i, 0))]
        + [pl.BlockSpec((D, D), lambda i: (0, 0))] * 3
        + [pl.BlockSpec((1, D), lambda i: (0, 0))],
        out_specs=pl.BlockSpec((BLK, D), lambda i: (i, 0)),
        out_shape=jax.ShapeDtypeStruct((NPAD, D), _f32),
    )(xp, x1, p, dinv2d, w0, w1, w2, b.reshape(1, D))


def kernel(x, edge_index, W, b):
    src = edge_index[0].astype(jnp.int32)
    dst = edge_index[1].astype(jnp.int32)
    src_p = jnp.concatenate(
        [src, jnp.zeros((EPAD - E,), jnp.int32)]).reshape(NCHUNKS, 1, CHUNK)
    dst_p = jnp.concatenate(
        [dst, jnp.full((EPAD - E,), TRASH, jnp.int32)]).reshape(NCHUNKS, 1, CHUNK)
    xp = jnp.pad(x, ((0, NPAD - N), (0, 0)))

    deg = _deg()(dst_p)
    dinv2d, h1 = _t1(deg.reshape(NPAD, 1), xp)
    agg1 = _prop()(src_p, dst_p, h1)
    x1, h2 = _t2(agg1, dinv2d)
    agg2 = _prop()(src_p, dst_p, h2)
    out = _matmul(xp, x1, agg2, dinv2d, W, b)
    return out[:N]


# speedup vs baseline: 4.7293x; 1.6842x over previous
import functools

import jax
import jax.numpy as jnp
from jax import lax
from jax.experimental import pallas as pl
from jax.experimental.pallas import tpu as pltpu
from jax.experimental.pallas import tpu_sc as plsc

N = 10000
D = 128
NC = 2
NS = 16
NW = NC * NS
NPB = 320
NPAD = NW * NPB
TRASH = NPAD
NALL = NPAD + 256
ZPW = NALL // NS
HALF = NPAD // 2
HALL = HALF + 128
HZPW = HALL // NS

E = 320000
CHUNK = 128
CPW = 79
NCHUNKS = NW * CPW
EPAD = NCHUNKS * CHUNK
CPW_DEG = NCHUNKS // NS

_f32 = jnp.float32


@functools.cache
def _mesh():
    return plsc.VectorSubcoreMesh(
        core_axis_name="c", subcore_axis_name="s", num_cores=NC, num_subcores=NS)


def _fill(ref, n, value):
    v = jnp.full((16,), value, _f32)
    for q in range(n // 16):
        ref[pl.ds(q * 16, 16)] = v


def _deg_body(dst_hbm, deg_hbm, deg_sp, dstv, onesv, zbuf, degv):
    cid = lax.axis_index("c")
    sid = lax.axis_index("s")
    gw = sid * NC + cid

    _fill(onesv, CHUNK, 1.0)
    _fill(zbuf, ZPW, 0.0)
    pltpu.sync_copy(zbuf, deg_sp.at[pl.ds(sid * ZPW, ZPW)])
    pltpu.sync_copy(dst_hbm.at[pl.ds(sid * CPW_DEG, CPW_DEG)], dstv)
    plsc.subcore_barrier()

    def deg_body(j, carry):
        pltpu.sync_copy(onesv, deg_sp.at[dstv.at[j, 0]], add=True)
        return carry

    lax.fori_loop(0, CPW_DEG, deg_body, 0)
    plsc.subcore_barrier()
    pltpu.sync_copy(deg_sp.at[pl.ds(gw * NPB, NPB)], degv)
    pltpu.sync_copy(degv, deg_hbm.at[pl.ds(gw * NPB, NPB)])


@functools.cache
def _deg():
    return pl.kernel(
        _deg_body,
        out_type=jax.ShapeDtypeStruct((NPAD,), _f32),
        mesh=_mesh(),
        scratch_types=[
            pltpu.VMEM_SHARED((NALL,), _f32),
            pltpu.VMEM((CPW_DEG, 1, CHUNK), jnp.int32),
            pltpu.VMEM((CHUNK,), _f32),
            pltpu.VMEM((ZPW,), _f32),
            pltpu.VMEM((NPB,), _f32),
        ],
    )


def _prop_body(src_hbm, dst_hbm, h_hbm, agg_hbm, agg_sp, srcv, dstv, rows,
               zrows):
    cid = lax.axis_index("c")
    sid = lax.axis_index("s")
    gw = sid * NC + cid

    for r in range(HZPW):
        for q in range(D // 16):
            zrows[r, pl.ds(q * 16, 16)] = jnp.zeros((16,), _f32)
    pltpu.sync_copy(zrows, agg_sp.at[pl.ds(sid * HZPW, HZPW)])
    plsc.subcore_barrier()

    lo = cid * HALF
    neg1 = jnp.full((16,), -1, jnp.int32)

    for p in range(2):
        boff = sid * CPW_DEG + p * CPW
        pltpu.sync_copy(src_hbm.at[pl.ds(boff, CPW)], srcv)
        pltpu.sync_copy(dst_hbm.at[pl.ds(boff, CPW)], dstv)

        def remap_body(j, carry):
            for q in range(CHUNK // 16):
                d = dstv[j, 0, pl.ds(q * 16, 16)] - lo
                keep = (d >= 0) & (d < HALF)
                dstv[j, 0, pl.ds(q * 16, 16)] = jnp.where(keep, d, neg1)
                s = srcv[j, 0, pl.ds(q * 16, 16)]
                srcv[j, 0, pl.ds(q * 16, 16)] = jnp.where(keep, s, neg1)
            return carry

        lax.fori_loop(0, CPW, remap_body, 0)

        def body(j, carry):
            pltpu.sync_copy(
                h_hbm.at[plsc.Indices(srcv.at[j, 0], ignored_value=-1)],
                rows)
            pltpu.sync_copy(
                rows,
                agg_sp.at[plsc.Indices(dstv.at[j, 0], ignored_value=-1)],
                add=True)
            return carry

        lax.fori_loop(0, CPW, body, 0)
    plsc.subcore_barrier()
    pltpu.sync_copy(agg_sp.at[pl.ds(sid * (HALF // NS), HALF // NS)],
                    zrows.at[pl.ds(0, HALF // NS)])
    pltpu.sync_copy(zrows.at[pl.ds(0, HALF // NS)],
                    agg_hbm.at[pl.ds(cid * HALF + sid * (HALF // NS),
                                     HALF // NS)])


@functools.cache
def _prop():
    return pl.kernel(
        _prop_body,
        out_type=jax.ShapeDtypeStruct((NPAD, D), _f32),
        mesh=_mesh(),
        scratch_types=[
            pltpu.VMEM_SHARED((HALL, D), _f32),
            pltpu.VMEM((CPW, 1, CHUNK), jnp.int32),
            pltpu.VMEM((CPW, 1, CHUNK), jnp.int32),
            pltpu.VMEM((CHUNK, D), _f32),
            pltpu.VMEM((HZPW, D), _f32),
        ],
    )


TBLK = 1024


def _t1_body(deg_ref, x_ref, dinv_ref, h1_ref):
    dinv = lax.rsqrt(jnp.maximum(deg_ref[...], 1.0))
    dinv_ref[...] = dinv
    h1_ref[...] = x_ref[...] * dinv


def _t1(deg2d, xp):
    return pl.pallas_call(
        _t1_body,
        grid=(NPAD // TBLK,),
        in_specs=[pl.BlockSpec((TBLK, 1), lambda i: (i, 0)),
                  pl.BlockSpec((TBLK, D), lambda i: (i, 0))],
        out_specs=[pl.BlockSpec((TBLK, 1), lambda i: (i, 0)),
                   pl.BlockSpec((TBLK, D), lambda i: (i, 0))],
        out_shape=(jax.ShapeDtypeStruct((NPAD, 1), _f32),
                   jax.ShapeDtypeStruct((NPAD, D), _f32)),
    )(deg2d, xp)


def _t2_body(p_ref, dinv_ref, x1_ref, h2_ref):
    dinv = dinv_ref[...]
    x1 = p_ref[...] * (-dinv)
    x1_ref[...] = x1
    h2_ref[...] = x1 * dinv


def _t2(p, dinv2d):
    return pl.pallas_call(
        _t2_body,
        grid=(NPAD // TBLK,),
        in_specs=[pl.BlockSpec((TBLK, D), lambda i: (i, 0)),
                  pl.BlockSpec((TBLK, 1), lambda i: (i, 0))],
        out_specs=[pl.BlockSpec((TBLK, D), lambda i: (i, 0)),
                   pl.BlockSpec((TBLK, D), lambda i: (i, 0))],
        out_shape=(jax.ShapeDtypeStruct((NPAD, D), _f32),
                   jax.ShapeDtypeStruct((NPAD, D), _f32)),
    )(p, dinv2d)


BLK = 512


def _mm_body(x_ref, x1_ref, p_ref, dinv_ref, w0_ref, w1_ref, w2_ref,
             b_ref, o_ref):
    x = x_ref[...]
    x2 = p_ref[...] * (-2.0 * dinv_ref[...]) - x
    acc = jnp.dot(x, w0_ref[...], preferred_element_type=_f32)
    acc += jnp.dot(x1_ref[...], w1_ref[...], preferred_element_type=_f32)
    acc += jnp.dot(x2, w2_ref[...], preferred_element_type=_f32)
    o_ref[...] = jnp.maximum(acc + b_ref[...], 0.0)


def _matmul(xp, x1, p, dinv2d, W, b):
    w0, w1, w2 = W[:D], W[D:2 * D], W[2 * D:]
    return pl.pallas_call(
        _mm_body,
        grid=(NPAD // BLK,),
        in_specs=[pl.BlockSpec((BLK, D), lambda i: (i, 0))] * 3
        + [pl.BlockSpec((BLK, 1), lambda i: (i, 0))]
        + [pl.BlockSpec((D, D), lambda i: (0, 0))] * 3
        + [pl.BlockSpec((1, D), lambda i: (0, 0))],
        out_specs=pl.BlockSpec((BLK, D), lambda i: (i, 0)),
        out_shape=jax.ShapeDtypeStruct((NPAD, D), _f32),
    )(xp, x1, p, dinv2d, w0, w1, w2, b.reshape(1, D))


def kernel(x, edge_index, W, b):
    src = edge_index[0].astype(jnp.int32)
    dst = edge_index[1].astype(jnp.int32)
    src_p = jnp.concatenate(
        [src, jnp.zeros((EPAD - E,), jnp.int32)]).reshape(NCHUNKS, 1, CHUNK)
    dst_p = jnp.concatenate(
        [dst, jnp.full((EPAD - E,), TRASH, jnp.int32)]).reshape(NCHUNKS, 1, CHUNK)
    xp = jnp.pad(x, ((0, NPAD - N), (0, 0)))

    deg = _deg()(dst_p)
    dinv2d, h1 = _t1(deg.reshape(NPAD, 1), xp)
    agg1 = _prop()(src_p, dst_p, h1)
    x1, h2 = _t2(agg1, dinv2d)
    agg2 = _prop()(src_p, dst_p, h2)
    out = _matmul(xp, x1, agg2, dinv2d, W, b)
    return out[:N]
